# baseline (speedup 1.0000x reference)
import functools

import jax
import jax.numpy as jnp
from jax import lax
from jax.experimental import pallas as pl
from jax.experimental.pallas import tpu as pltpu
from jax.experimental.pallas import tpu_sc as plsc

B, L, H = 128, 8192, 64
K = 8


_BB, _LB = 8, 4096


def _scores_body(w_ref, h_ref, o_ref):
    w = w_ref[...]
    for bb in range(_BB):
        ht = h_ref[bb]
        s = lax.dot_general(
            w, ht, (((1,), (0,)), ((), ())),
            preferred_element_type=jnp.float32,
        )
        o_ref[bb, :] = s[0]


_CB = 64


def _scores_body2(w_ref, h1_ref, h2_ref, o1_ref, o2_ref):
    w = w_ref[...]
    for bb in range(_BB):
        o1_ref[bb, :] = lax.dot_general(
            w, h1_ref[bb], (((1,), (0,)), ((), ())),
            preferred_element_type=jnp.float32)[0]
        o2_ref[bb, :] = lax.dot_general(
            w, h2_ref[bb], (((1,), (0,)), ((), ())),
            preferred_element_type=jnp.float32)[0]


def _scores_tc(ht, W):
    grid = (_CB // _BB, L // _LB)
    return pl.pallas_call(
        _scores_body2,
        grid=grid,
        in_specs=[
            pl.BlockSpec((1, H), lambda i, j: (0, 0)),
            pl.BlockSpec((_BB, H, _LB), lambda i, j: (i, 0, j)),
            pl.BlockSpec((_BB, H, _LB), lambda i, j: (i + _CB // _BB, 0, j)),
        ],
        out_specs=[
            pl.BlockSpec((_BB, _LB), lambda i, j: (i, j)),
            pl.BlockSpec((_BB, _LB), lambda i, j: (i, j)),
        ],
        out_shape=[
            jax.ShapeDtypeStruct((_CB, L), jnp.float32),
            jax.ShapeDtypeStruct((_CB, L), jnp.float32),
        ],
    )(W, ht, ht)


_NC, _NS, _LN = 2, 16, 16
_NW = _NC * _NS
_R = _CB // _NW
_NCHUNK = L // _LN
_GRP = 8
_CAND = 64
_NEG = -3.0e38
_BIG = 2 ** 30


def _topk_body(scores_hbm, idx_hbm, sc_v, cv_v, ci_v, ix_v, sem):
    wid = lax.axis_index("s") * _NC + lax.axis_index("c")
    base = wid * _R
    pltpu.sync_copy(scores_hbm.at[pl.ds(base, _R)], sc_v)

    lane = lax.broadcasted_iota(jnp.int32, (_LN,), 0)
    neg16 = jnp.full((_LN,), _NEG, jnp.float32)
    big16 = jnp.full((_LN,), _BIG, jnp.int32)
    zeros16 = jnp.zeros((_LN,), jnp.int32)

    for r in range(_R):
        def step_a(g, accs, r=r):
            out = []
            for u in range(_GRP):
                off = pl.multiple_of(g * (_GRP * _LN) + u * _LN, _LN)
                out.append(jnp.maximum(accs[u], sc_v[r, pl.ds(off, _LN)]))
            return tuple(out)

        accs = lax.fori_loop(0, _NCHUNK // _GRP, step_a, (neg16,) * _GRP)
        acc = accs[0]
        for u in range(1, _GRP):
            acc = jnp.maximum(acc, accs[u])
        t0 = plsc.sort_key_val(acc, acc, descending=True)[0][K - 1]

        for j in range(_CAND // _LN):
            cv_v[pl.ds(j * _LN, _LN)] = neg16
            ci_v[pl.ds(j * _LN, _LN)] = big16

        def step_b(g, cnt, r=r):
            gbase = g * (_GRP * _LN)
            vs = []
            for u in range(_GRP):
                off = pl.multiple_of(gbase + u * _LN, _LN)
                vs.append(sc_v[r, pl.ds(off, _LN)])
            gmax = vs[0]
            for u in range(1, _GRP):
                gmax = jnp.maximum(gmax, vs[u])
            hit = plsc.all_reduce_population_count(gmax >= t0)[0] > 0

            def heavy(cnt):
                c = cnt
                for u in range(_GRP):
                    m = vs[u] >= t0
                    pos = plsc.cumsum(m.astype(jnp.int32)) - 1 + c
                    pos = jnp.minimum(pos, _CAND - 1)
                    plsc.store_scatter(cv_v, [pos], vs[u], mask=m)
                    plsc.store_scatter(
                        ci_v, [pos], lane + (gbase + u * _LN), mask=m)
                    c = c + plsc.all_reduce_population_count(m)
                return c

            return lax.cond(hit, heavy, lambda c: c, cnt)

        lax.fori_loop(0, _NCHUNK // _GRP, step_b, zeros16)

        cvs = [cv_v[pl.ds(j * _LN, _LN)] for j in range(_CAND // _LN)]
        cis = [ci_v[pl.ds(j * _LN, _LN)] for j in range(_CAND // _LN)]
        w_idx = zeros16
        for k in range(K):
            mall = cvs[0]
            for j in range(1, len(cvs)):
                mall = jnp.maximum(mall, cvs[j])
            t = jnp.max(mall)
            wi = jnp.min(jnp.where(cvs[0] == t, cis[0], _BIG))
            for j in range(1, len(cvs)):
                wi = jnp.minimum(wi, jnp.min(jnp.where(cvs[j] == t, cis[j], _BIG)))
            cvs = [jnp.where(cis[j] == wi, _NEG, cvs[j]) for j in range(len(cvs))]
            w_idx = jnp.where(lane == k, wi, w_idx)

        ix_v[r, :] = w_idx

    pltpu.sync_copy(ix_v, idx_hbm.at[pl.ds(base, _R)])


def _topk_sc(scores):
    mesh = plsc.VectorSubcoreMesh(core_axis_name="c", subcore_axis_name="s")
    fn = pl.kernel(
        _topk_body,
        out_type=jax.ShapeDtypeStruct((_CB, _LN), jnp.int32),
        mesh=mesh,
        scratch_types=[
            pltpu.VMEM((_R, L), jnp.float32),
            pltpu.VMEM((_CAND,), jnp.float32),
            pltpu.VMEM((_CAND,), jnp.int32),
            pltpu.VMEM((_R, _LN), jnp.int32),
            pltpu.SemaphoreType.DMA,
        ],
        compiler_params=pltpu.CompilerParams(needs_layout_passes=False),
    )
    return fn(scores)


_MB = 16


def _mask_body(idx_ref, o_ref):
    idx = idx_ref[...]
    li = lax.broadcasted_iota(jnp.int32, (_MB, L), 1)
    m = li == idx[:, 0:1]
    for k in range(1, K):
        m = m | (li == idx[:, k:k + 1])
    o_ref[...] = m


def _mask_tc(idx):
    grid = (B // _MB,)
    return pl.pallas_call(
        _mask_body,
        grid=grid,
        in_specs=[pl.BlockSpec((_MB, _LN), lambda i: (i, 0))],
        out_specs=pl.BlockSpec((_MB, L), lambda i: (i, 0)),
        out_shape=jax.ShapeDtypeStruct((B, L), jnp.bool_),
    )(idx)


def kernel(hidden, attn, loss_per_token, W, b):
    ht = jnp.swapaxes(hidden, 1, 2)
    s1, s2 = _scores_tc(ht, W)
    i1 = _topk_sc(s1)
    i2 = _topk_sc(s2)
    return _mask_tc(jnp.concatenate([i1, i2], axis=0))


# speedup vs baseline: 1.0506x; 1.0506x over previous
import functools

import jax
import jax.numpy as jnp
from jax import lax
from jax.experimental import pallas as pl
from jax.experimental.pallas import tpu as pltpu
from jax.experimental.pallas import tpu_sc as plsc

B, L, H = 128, 8192, 64
K = 8


_BB, _LB = 8, 8192


_CB = B


def _scores_body(w_ref, h_ref, o_ref):
    w = w_ref[...]
    for bb in range(_BB):
        o_ref[bb, :] = lax.dot_general(
            w, h_ref[bb], (((1,), (0,)), ((), ())),
            preferred_element_type=jnp.float32)[0]


def _scores_tc(ht, W):
    grid = (B // _BB,)
    return pl.pallas_call(
        _scores_body,
        grid=grid,
        in_specs=[
            pl.BlockSpec((1, H), lambda i: (0, 0)),
            pl.BlockSpec((_BB, H, _LB), lambda i: (i, 0, 0)),
        ],
        out_specs=pl.BlockSpec((_BB, _LB), lambda i: (i, 0)),
        out_shape=jax.ShapeDtypeStruct((B, L), jnp.float32),
    )(W, ht)


_NC, _NS, _LN = 2, 16, 16
_NW = _NC * _NS
_R = _CB // _NW
_NCHUNK = L // _LN
_GRP = 8
_CAND = 64
_NEG = -3.0e38
_BIG = 2 ** 30


def _topk_body(scores_hbm, idx_hbm, sc_v, cv_v, ci_v, ix_v, sem):
    wid = lax.axis_index("s") * _NC + lax.axis_index("c")
    base = wid * _R
    pltpu.sync_copy(scores_hbm.at[pl.ds(base, _R)], sc_v)

    lane = lax.broadcasted_iota(jnp.int32, (_LN,), 0)
    neg16 = jnp.full((_LN,), _NEG, jnp.float32)
    big16 = jnp.full((_LN,), _BIG, jnp.int32)
    zeros16 = jnp.zeros((_LN,), jnp.int32)

    for r in range(_R):
        def step_a(g, accs, r=r):
            out = []
            for u in range(_GRP):
                off = pl.multiple_of(g * (_GRP * _LN) + u * _LN, _LN)
                out.append(jnp.maximum(accs[u], sc_v[r, pl.ds(off, _LN)]))
            return tuple(out)

        accs = lax.fori_loop(0, _NCHUNK // _GRP, step_a, (neg16,) * _GRP)
        acc = accs[0]
        for u in range(1, _GRP):
            acc = jnp.maximum(acc, accs[u])
        t0 = plsc.sort_key_val(acc, acc, descending=True)[0][K - 1]

        for j in range(_CAND // _LN):
            cv_v[pl.ds(j * _LN, _LN)] = neg16
            ci_v[pl.ds(j * _LN, _LN)] = big16

        def step_b(g, cnt, r=r):
            gbase = g * (_GRP * _LN)
            vs = []
            for u in range(_GRP):
                off = pl.multiple_of(gbase + u * _LN, _LN)
                vs.append(sc_v[r, pl.ds(off, _LN)])
            gmax = vs[0]
            for u in range(1, _GRP):
                gmax = jnp.maximum(gmax, vs[u])
            hit = plsc.all_reduce_population_count(gmax >= t0)[0] > 0

            def heavy(cnt):
                c = cnt
                for u in range(_GRP):
                    m = vs[u] >= t0
                    pos = plsc.cumsum(m.astype(jnp.int32)) - 1 + c
                    pos = jnp.minimum(pos, _CAND - 1)
                    plsc.store_scatter(cv_v, [pos], vs[u], mask=m)
                    plsc.store_scatter(
                        ci_v, [pos], lane + (gbase + u * _LN), mask=m)
                    c = c + plsc.all_reduce_population_count(m)
                return c

            return lax.cond(hit, heavy, lambda c: c, cnt)

        lax.fori_loop(0, _NCHUNK // _GRP, step_b, zeros16)

        cvs = [cv_v[pl.ds(j * _LN, _LN)] for j in range(_CAND // _LN)]
        cis = [ci_v[pl.ds(j * _LN, _LN)] for j in range(_CAND // _LN)]
        w_idx = zeros16
        for k in range(K):
            mall = cvs[0]
            for j in range(1, len(cvs)):
                mall = jnp.maximum(mall, cvs[j])
            t = jnp.max(mall)
            wi = jnp.min(jnp.where(cvs[0] == t, cis[0], _BIG))
            for j in range(1, len(cvs)):
                wi = jnp.minimum(wi, jnp.min(jnp.where(cvs[j] == t, cis[j], _BIG)))
            cvs = [jnp.where(cis[j] == wi, _NEG, cvs[j]) for j in range(len(cvs))]
            w_idx = jnp.where(lane == k, wi, w_idx)

        ix_v[r, :] = w_idx

    pltpu.sync_copy(ix_v, idx_hbm.at[pl.ds(base, _R)])


def _topk_sc(scores):
    mesh = plsc.VectorSubcoreMesh(core_axis_name="c", subcore_axis_name="s")
    fn = pl.kernel(
        _topk_body,
        out_type=jax.ShapeDtypeStruct((_CB, _LN), jnp.int32),
        mesh=mesh,
        scratch_types=[
            pltpu.VMEM((_R, L), jnp.float32),
            pltpu.VMEM((_CAND,), jnp.float32),
            pltpu.VMEM((_CAND,), jnp.int32),
            pltpu.VMEM((_R, _LN), jnp.int32),
            pltpu.SemaphoreType.DMA,
        ],
        compiler_params=pltpu.CompilerParams(needs_layout_passes=False),
    )
    return fn(scores)


_MB = 16


def _mask_body(idx_ref, o_ref):
    idx = idx_ref[...]
    li = lax.broadcasted_iota(jnp.int32, (_MB, L), 1)
    m = li == idx[:, 0:1]
    for k in range(1, K):
        m = m | (li == idx[:, k:k + 1])
    o_ref[...] = m


def _mask_tc(idx):
    grid = (B // _MB,)
    return pl.pallas_call(
        _mask_body,
        grid=grid,
        in_specs=[pl.BlockSpec((_MB, _LN), lambda i: (i, 0))],
        out_specs=pl.BlockSpec((_MB, L), lambda i: (i, 0)),
        out_shape=jax.ShapeDtypeStruct((B, L), jnp.bool_),
    )(idx)


def kernel(hidden, attn, loss_per_token, W, b):
    ht = jnp.swapaxes(hidden, 1, 2)
    scores = _scores_tc(ht, W)
    idx = _topk_sc(scores)
    return _mask_tc(idx)


# speedup vs baseline: 1.0616x; 1.0105x over previous
import functools

import jax
import jax.numpy as jnp
from jax import lax
from jax.experimental import pallas as pl
from jax.experimental.pallas import tpu as pltpu
from jax.experimental.pallas import tpu_sc as plsc

B, L, H = 128, 8192, 64
K = 8


_BB, _LB = 8, 8192


_CB = B


def _scores_body(w_ref, h_ref, o_ref):
    w = w_ref[...]
    for bb in range(_BB):
        o_ref[bb, :] = lax.dot_general(
            w, h_ref[bb], (((1,), (0,)), ((), ())),
            preferred_element_type=jnp.float32)[0]


def _scores_tc(ht, W):
    grid = (B // _BB,)
    return pl.pallas_call(
        _scores_body,
        grid=grid,
        in_specs=[
            pl.BlockSpec((1, H), lambda i: (0, 0)),
            pl.BlockSpec((_BB, H, _LB), lambda i: (i, 0, 0)),
        ],
        out_specs=pl.BlockSpec((_BB, _LB), lambda i: (i, 0)),
        out_shape=jax.ShapeDtypeStruct((B, L), jnp.float32),
    )(W, ht)


_NC, _NS, _LN = 2, 16, 16
_NW = _NC * _NS
_R = _CB // _NW
_NCHUNK = L // _LN
_GRP = 8
_CAND = 64
_NEG = -3.0e38
_BIG = 2 ** 30


def _topk_body(scores_hbm, idx_hbm, sc_v, cv_v, ci_v, ix_v, sem):
    wid = lax.axis_index("s") * _NC + lax.axis_index("c")
    base = wid * _R
    copies = [
        pltpu.async_copy(scores_hbm.at[pl.ds(base + r, 1)],
                         sc_v.at[pl.ds(r, 1)], sem)
        for r in range(_R)
    ]

    lane = lax.broadcasted_iota(jnp.int32, (_LN,), 0)
    neg16 = jnp.full((_LN,), _NEG, jnp.float32)
    big16 = jnp.full((_LN,), _BIG, jnp.int32)
    zeros16 = jnp.zeros((_LN,), jnp.int32)

    for r in range(_R):
        copies[r].wait()
        def step_a(g, accs, r=r):
            out = []
            for u in range(_GRP):
                off = pl.multiple_of(g * (_GRP * _LN) + u * _LN, _LN)
                out.append(jnp.maximum(accs[u], sc_v[r, pl.ds(off, _LN)]))
            return tuple(out)

        accs = lax.fori_loop(0, _NCHUNK // _GRP, step_a, (neg16,) * _GRP)
        acc = accs[0]
        for u in range(1, _GRP):
            acc = jnp.maximum(acc, accs[u])
        t0 = plsc.sort_key_val(acc, acc, descending=True)[0][K - 1]

        for j in range(_CAND // _LN):
            cv_v[pl.ds(j * _LN, _LN)] = neg16
            ci_v[pl.ds(j * _LN, _LN)] = big16

        def step_b(g, cnt, r=r):
            gbase = g * (_GRP * _LN)
            vs = []
            for u in range(_GRP):
                off = pl.multiple_of(gbase + u * _LN, _LN)
                vs.append(sc_v[r, pl.ds(off, _LN)])
            gmax = vs[0]
            for u in range(1, _GRP):
                gmax = jnp.maximum(gmax, vs[u])
            hit = plsc.all_reduce_population_count(gmax >= t0)[0] > 0

            def heavy(cnt):
                c = cnt
                for u in range(_GRP):
                    m = vs[u] >= t0
                    pos = plsc.cumsum(m.astype(jnp.int32)) - 1 + c
                    pos = jnp.minimum(pos, _CAND - 1)
                    plsc.store_scatter(cv_v, [pos], vs[u], mask=m)
                    plsc.store_scatter(
                        ci_v, [pos], lane + (gbase + u * _LN), mask=m)
                    c = c + plsc.all_reduce_population_count(m)
                return c

            return lax.cond(hit, heavy, lambda c: c, cnt)

        lax.fori_loop(0, _NCHUNK // _GRP, step_b, zeros16)

        cvs = [cv_v[pl.ds(j * _LN, _LN)] for j in range(_CAND // _LN)]
        cis = [ci_v[pl.ds(j * _LN, _LN)] for j in range(_CAND // _LN)]
        w_idx = zeros16
        for k in range(K):
            mall = cvs[0]
            for j in range(1, len(cvs)):
                mall = jnp.maximum(mall, cvs[j])
            t = jnp.max(mall)
            wi = jnp.min(jnp.where(cvs[0] == t, cis[0], _BIG))
            for j in range(1, len(cvs)):
                wi = jnp.minimum(wi, jnp.min(jnp.where(cvs[j] == t, cis[j], _BIG)))
            cvs = [jnp.where(cis[j] == wi, _NEG, cvs[j]) for j in range(len(cvs))]
            w_idx = jnp.where(lane == k, wi, w_idx)

        ix_v[r, :] = w_idx

    pltpu.sync_copy(ix_v, idx_hbm.at[pl.ds(base, _R)])


def _topk_sc(scores):
    mesh = plsc.VectorSubcoreMesh(core_axis_name="c", subcore_axis_name="s")
    fn = pl.kernel(
        _topk_body,
        out_type=jax.ShapeDtypeStruct((_CB, _LN), jnp.int32),
        mesh=mesh,
        scratch_types=[
            pltpu.VMEM((_R, L), jnp.float32),
            pltpu.VMEM((_CAND,), jnp.float32),
            pltpu.VMEM((_CAND,), jnp.int32),
            pltpu.VMEM((_R, _LN), jnp.int32),
            pltpu.SemaphoreType.DMA,
        ],
        compiler_params=pltpu.CompilerParams(needs_layout_passes=False),
    )
    return fn(scores)


_MB = 16


def _mask_body(idx_ref, o_ref):
    idx = idx_ref[...].astype(jnp.int16)
    li = lax.broadcasted_iota(jnp.int16, (_MB, L), 1)
    m = li == idx[:, 0:1]
    for k in range(1, K):
        m = m | (li == idx[:, k:k + 1])
    o_ref[...] = m


def _mask_tc(idx):
    grid = (B // _MB,)
    return pl.pallas_call(
        _mask_body,
        grid=grid,
        in_specs=[pl.BlockSpec((_MB, _LN), lambda i: (i, 0))],
        out_specs=pl.BlockSpec((_MB, L), lambda i: (i, 0)),
        out_shape=jax.ShapeDtypeStruct((B, L), jnp.bool_),
    )(idx)


def kernel(hidden, attn, loss_per_token, W, b):
    ht = jnp.swapaxes(hidden, 1, 2)
    scores = _scores_tc(ht, W)
    idx = _topk_sc(scores)
    return _mask_tc(idx)
